# Initial kernel scaffold; baseline (speedup 1.0000x reference)
import jax
import jax.numpy as jnp
from jax.experimental import pallas as pl


def kernel(x, edge_index, edge_attr, W1, b1, g1, be1, W2, b2, g2, be2, W3, b3, Wl, bl):
    raise NotImplementedError("write your pallas kernel here")


# speedup vs baseline: 25.5428x; 25.5428x over previous
import functools

import jax
import jax.numpy as jnp
from jax import lax
from jax.experimental import pallas as pl
from jax.experimental.pallas import tpu as pltpu
from jax.experimental.pallas import tpu_sc as plsc

CH = 128
EPS = 1e-5
F1 = 16


def _rsqrt_newton(d):
    ib = plsc.bitcast(d, jnp.int32)
    y = plsc.bitcast(jnp.int32(0x5F3759DF) - (ib >> 1), jnp.float32)
    for _ in range(3):
        y = y * (1.5 - 0.5 * d * y * y)
    return y


def _make_precompute(rows, rpt, npad, nt):
    mesh = plsc.VectorSubcoreMesh(core_axis_name="c", subcore_axis_name="s")

    @functools.partial(
        pl.kernel,
        mesh=mesh,
        out_type=jax.ShapeDtypeStruct((rows, CH), jnp.float32),
        compiler_params=pltpu.CompilerParams(needs_layout_passes=False),
        scratch_types=[
            pltpu.VMEM((rpt, CH), jnp.int32),
            pltpu.VMEM((rpt, CH), jnp.int32),
            pltpu.VMEM((rpt, CH), jnp.float32),
            pltpu.VMEM((rpt, CH), jnp.float32),
            pltpu.VMEM((npad,), jnp.float32),
            pltpu.VMEM((nt,), jnp.float32),
            pltpu.VMEM_SHARED((npad,), jnp.float32),
            pltpu.VMEM_SHARED((npad,), jnp.float32),
        ],
    )
    def pre(src_hbm, dst_hbm, w_hbm, norm_hbm,
            srcb, dstb, wb, normb, dinvb, degb, deg_sp, dinv_sp):
        cid = lax.axis_index("c")
        sid = lax.axis_index("s")

        @pl.when(cid == 0)
        def _stage():
            t0 = sid * rpt
            pltpu.sync_copy(src_hbm.at[pl.ds(t0, rpt)], srcb)
            pltpu.sync_copy(dst_hbm.at[pl.ds(t0, rpt)], dstb)
            pltpu.sync_copy(w_hbm.at[pl.ds(t0, rpt)], wb)

            def zb(i, carry):
                degb[pl.ds(i * 16, 16)] = jnp.zeros((16,), jnp.float32)
                return carry
            lax.fori_loop(0, nt // 16, zb, 0)
            pltpu.sync_copy(degb, deg_sp.at[pl.ds(sid * nt, nt)])

        plsc.subcore_barrier()

        @pl.when(cid == 0)
        def _deg():
            def sc(j, carry):
                pltpu.sync_copy(wb.at[j], deg_sp.at[dstb.at[j]], add=True)
                return carry
            lax.fori_loop(0, rpt, sc, 0)

        plsc.subcore_barrier()

        @pl.when(cid == 0)
        def _dinv():
            pltpu.sync_copy(deg_sp.at[pl.ds(sid * nt, nt)], degb)

            def nw(i, carry):
                d = degb[pl.ds(i * 16, 16)]
                degb[pl.ds(i * 16, 16)] = _rsqrt_newton(d)
                return carry
            lax.fori_loop(0, nt // 16, nw, 0)
            pltpu.sync_copy(degb, dinv_sp.at[pl.ds(sid * nt, nt)])

        plsc.subcore_barrier()

        @pl.when(cid == 0)
        def _norm():
            pltpu.sync_copy(dinv_sp, dinvb)

            def nc(j, carry):
                for g in range(CH // 16):
                    s16 = srcb[j, pl.ds(g * 16, 16)]
                    d16 = dstb[j, pl.ds(g * 16, 16)]
                    w16 = wb[j, pl.ds(g * 16, 16)]
                    nv = (plsc.load_gather(dinvb, [s16]) * w16
                          * plsc.load_gather(dinvb, [d16]))
                    normb[j, pl.ds(g * 16, 16)] = nv
                return carry
            lax.fori_loop(0, rpt, nc, 0)
            pltpu.sync_copy(normb, norm_hbm.at[pl.ds(sid * rpt, rpt)])

    return pre


def _make_agg(rows, rpt, npad, nt):
    mesh = plsc.VectorSubcoreMesh(core_axis_name="c", subcore_axis_name="s")

    @functools.partial(
        pl.kernel,
        mesh=mesh,
        out_type=jax.ShapeDtypeStruct((2 * npad, F1), jnp.float32),
        compiler_params=pltpu.CompilerParams(
            needs_layout_passes=False, use_tc_tiling_on_sc=False),
        scratch_types=[
            pltpu.VMEM((rpt, CH), jnp.int32),
            pltpu.VMEM((rpt, CH), jnp.int32),
            pltpu.VMEM((rpt, CH), jnp.float32),
            pltpu.VMEM((CH, F1), jnp.float32),
            pltpu.VMEM((nt, F1), jnp.float32),
            pltpu.VMEM_SHARED((npad, F1), jnp.float32),
        ],
    )
    def agg(h_hbm, src_hbm, dst_hbm, norm_hbm, out_hbm,
            srcb, dstb, normb, hbuf, zbuf, acc_sp):
        cid = lax.axis_index("c")
        sid = lax.axis_index("s")
        wid = cid * 16 + sid
        r0 = wid * rpt
        pltpu.sync_copy(src_hbm.at[pl.ds(r0, rpt)], srcb)
        pltpu.sync_copy(dst_hbm.at[pl.ds(r0, rpt)], dstb)
        pltpu.sync_copy(norm_hbm.at[pl.ds(r0, rpt)], normb)

        def zb(i, carry):
            zbuf[i, :] = jnp.zeros((F1,), jnp.float32)
            return carry
        lax.fori_loop(0, nt, zb, 0)
        pltpu.sync_copy(zbuf, acc_sp.at[pl.ds(sid * nt, nt)])
        plsc.subcore_barrier()

        def chunk(j, carry):
            pltpu.sync_copy(h_hbm.at[srcb.at[j]], hbuf)
            for g in range(CH // 16):
                n16 = normb[j, pl.ds(g * 16, 16)]
                for l in range(16):
                    r = g * 16 + l
                    hbuf[r, :] = hbuf[r, :] * n16[l]
            pltpu.sync_copy(hbuf, acc_sp.at[dstb.at[j]], add=True)
            return carry
        lax.fori_loop(0, rpt, chunk, 0)
        plsc.subcore_barrier()
        pltpu.sync_copy(acc_sp.at[pl.ds(sid * nt, nt)],
                        out_hbm.at[pl.ds(cid * npad + sid * nt, nt)])

    return agg


def _mm1_body(x_ref, w_ref, o_ref):
    o_ref[...] = jnp.dot(x_ref[...], w_ref[...],
                         preferred_element_type=jnp.float32)


def _ew1_body(p_ref, s_ref, c_ref, o_ref):
    o_ref[...] = jnp.maximum(
        (p_ref[0] + p_ref[1]) * s_ref[...] + c_ref[...], 0.0)


def _ew2_body(p_ref, w2_ref, s2_ref, c2_ref, w3_ref, o_ref):
    t = p_ref[0] + p_ref[1]
    u = jnp.maximum(
        jnp.dot(t, w2_ref[...], preferred_element_type=jnp.float32)
        * s2_ref[...] + c2_ref[...], 0.0)
    h3 = jnp.dot(u, w3_ref[...], preferred_element_type=jnp.float32)
    o_ref[...] = jnp.concatenate([h3, jnp.zeros_like(h3)], axis=1)


def _ew3_body(p_ref, b3_ref, wl_ref, bl_ref, o_ref):
    t = (p_ref[0] + p_ref[1])[:, :8] + b3_ref[...]
    o_ref[...] = jax.nn.sigmoid(
        jnp.dot(t, wl_ref[...], preferred_element_type=jnp.float32)
        + bl_ref[...])


def _tc(body, out_shape, *args):
    return pl.pallas_call(
        body, out_shape=jax.ShapeDtypeStruct(out_shape, jnp.float32))(*args)


def kernel(x, edge_index, edge_attr, W1, b1, g1, be1, W2, b2, g2, be2,
           W3, b3, Wl, bl):
    f32 = jnp.float32
    n, _ = x.shape
    e = edge_index.shape[1]

    nt = -(-n // 256) * 16
    npad = 16 * nt
    ef = -(-(e + n) // (256 * CH)) * (256 * CH)
    rows = ef // CH
    rpt32 = rows // 32
    rpt16 = rows // 16
    pad = ef - e - n

    idt = edge_index.dtype
    loops = jnp.arange(n, dtype=idt)
    zpad_i = n + (jnp.arange(pad, dtype=idt) % (npad - n))
    src2 = jnp.concatenate([edge_index[0], loops, zpad_i]).reshape(rows, CH)
    dst2 = jnp.concatenate([edge_index[1], loops, zpad_i]).reshape(rows, CH)
    w2d = jnp.concatenate(
        [edge_attr, jnp.ones((n,), f32), jnp.zeros((pad,), f32)]
    ).reshape(rows, CH)
    xp = jnp.pad(x, ((0, npad - n), (0, 0)))

    bn_s = 1.0 / jnp.sqrt(jnp.float32(1.0 + EPS))
    s1 = (g1 * bn_s).reshape(1, F1)
    c1 = (b1 * g1 * bn_s + be1).reshape(1, F1)
    s2 = (g2 * bn_s).reshape(1, 4)
    c2 = (b2 * g2 * bn_s + be2).reshape(1, 4)

    norm2 = _make_precompute(rows, rpt16, npad, nt)(src2, dst2, w2d)

    agg = _make_agg(rows, rpt32, npad, nt)

    h1 = _tc(_mm1_body, (npad, F1), xp, W1.T)
    p1 = agg(h1, src2, dst2, norm2).reshape(2, npad, F1)
    z1 = _tc(_ew1_body, (npad, F1), p1, s1, c1)
    p2 = agg(z1, src2, dst2, norm2).reshape(2, npad, F1)
    h3 = _tc(_ew2_body, (npad, F1), p2, W2.T, s2, c2, W3.T)
    p3 = agg(h3, src2, dst2, norm2).reshape(2, npad, F1)
    y = _tc(_ew3_body, (npad, 8), p3, b3.reshape(1, 8), Wl.T, bl.reshape(1, 8))
    return y[:n]


# speedup vs baseline: 47.0410x; 1.8417x over previous
import functools

import jax
import jax.numpy as jnp
from jax import lax
from jax.experimental import pallas as pl
from jax.experimental.pallas import tpu as pltpu
from jax.experimental.pallas import tpu_sc as plsc

CH = 128
EPS = 1e-5
F1 = 16


def _rsqrt_newton(d):
    ib = plsc.bitcast(d, jnp.int32)
    y = plsc.bitcast(jnp.int32(0x5F3759DF) - (ib >> 1), jnp.float32)
    for _ in range(3):
        y = y * (1.5 - 0.5 * d * y * y)
    return y


def _make_precompute(rows, rpt, npad, nt):
    mesh = plsc.VectorSubcoreMesh(core_axis_name="c", subcore_axis_name="s")

    @functools.partial(
        pl.kernel,
        mesh=mesh,
        out_type=jax.ShapeDtypeStruct((rows, CH), jnp.float32),
        compiler_params=pltpu.CompilerParams(needs_layout_passes=False),
        scratch_types=[
            pltpu.VMEM((rpt, CH), jnp.int32),
            pltpu.VMEM((rpt, CH), jnp.int32),
            pltpu.VMEM((rpt, CH), jnp.float32),
            pltpu.VMEM((rpt, CH), jnp.float32),
            pltpu.VMEM((npad,), jnp.float32),
            pltpu.VMEM((nt,), jnp.float32),
            pltpu.VMEM_SHARED((npad,), jnp.float32),
            pltpu.VMEM_SHARED((npad,), jnp.float32),
            pltpu.SemaphoreType.DMA,
        ],
    )
    def pre(src_hbm, dst_hbm, w_hbm, norm_hbm,
            srcb, dstb, wb, normb, dinvb, degb, deg_sp, dinv_sp, dsem):
        cid = lax.axis_index("c")
        sid = lax.axis_index("s")

        @pl.when(cid == 0)
        def _stage():
            t0 = sid * rpt
            pltpu.sync_copy(src_hbm.at[pl.ds(t0, rpt)], srcb)
            pltpu.sync_copy(dst_hbm.at[pl.ds(t0, rpt)], dstb)
            pltpu.sync_copy(w_hbm.at[pl.ds(t0, rpt)], wb)

            def zb(i, carry):
                degb[pl.ds(i * 16, 16)] = jnp.zeros((16,), jnp.float32)
                return carry
            lax.fori_loop(0, nt // 16, zb, 0)
            pltpu.sync_copy(degb, deg_sp.at[pl.ds(sid * nt, nt)])

        plsc.subcore_barrier()

        @pl.when(cid == 0)
        def _deg():
            k = 8
            def grp(g, carry):
                for b in range(k):
                    pltpu.async_copy(wb.at[g * k + b],
                                     deg_sp.at[dstb.at[g * k + b]], dsem,
                                     add=True)
                for _ in range(k):
                    pltpu.make_async_copy(
                        wb.at[0], deg_sp.at[dstb.at[0]], dsem).wait()
                return carry
            lax.fori_loop(0, rpt // k, grp, 0)

        plsc.subcore_barrier()

        @pl.when(cid == 0)
        def _dinv():
            pltpu.sync_copy(deg_sp.at[pl.ds(sid * nt, nt)], degb)

            def nw(i, carry):
                d = degb[pl.ds(i * 16, 16)]
                degb[pl.ds(i * 16, 16)] = _rsqrt_newton(d)
                return carry
            lax.fori_loop(0, nt // 16, nw, 0)
            pltpu.sync_copy(degb, dinv_sp.at[pl.ds(sid * nt, nt)])

        plsc.subcore_barrier()

        @pl.when(cid == 0)
        def _norm():
            pltpu.sync_copy(dinv_sp, dinvb)

            def nc(j, carry):
                for g in range(CH // 16):
                    s16 = srcb[j, pl.ds(g * 16, 16)]
                    d16 = dstb[j, pl.ds(g * 16, 16)]
                    w16 = wb[j, pl.ds(g * 16, 16)]
                    nv = (plsc.load_gather(dinvb, [s16]) * w16
                          * plsc.load_gather(dinvb, [d16]))
                    normb[j, pl.ds(g * 16, 16)] = nv
                return carry
            lax.fori_loop(0, rpt, nc, 0)
            pltpu.sync_copy(normb, norm_hbm.at[pl.ds(sid * rpt, rpt)])

    return pre


def _make_agg(rows, rpt, npad, nt):
    mesh = plsc.VectorSubcoreMesh(core_axis_name="c", subcore_axis_name="s")

    nb = 4

    @functools.partial(
        pl.kernel,
        mesh=mesh,
        out_type=jax.ShapeDtypeStruct((2 * npad, F1), jnp.float32),
        compiler_params=pltpu.CompilerParams(
            needs_layout_passes=False, use_tc_tiling_on_sc=False),
        scratch_types=[
            pltpu.VMEM((rpt, CH), jnp.int32),
            pltpu.VMEM((rpt, CH), jnp.int32),
            pltpu.VMEM((rpt, CH), jnp.float32),
            pltpu.VMEM((nb, CH, F1), jnp.float32),
            pltpu.VMEM((nb, CH, F1), jnp.float32),
            pltpu.VMEM((nt, F1), jnp.float32),
            pltpu.VMEM_SHARED((npad, F1), jnp.float32),
            pltpu.SemaphoreType.DMA((nb,)),
            pltpu.SemaphoreType.DMA((nb,)),
        ],
    )
    def agg(h_hbm, src_hbm, dst_hbm, norm_hbm, out_hbm,
            srcb, dstb, normb, hbuf, sbuf, zbuf, acc_sp, gsem, ssem):
        cid = lax.axis_index("c")
        sid = lax.axis_index("s")
        wid = cid * 16 + sid
        r0 = wid * rpt
        pltpu.sync_copy(src_hbm.at[pl.ds(r0, rpt)], srcb)
        pltpu.sync_copy(dst_hbm.at[pl.ds(r0, rpt)], dstb)
        pltpu.sync_copy(norm_hbm.at[pl.ds(r0, rpt)], normb)

        def zb(i, carry):
            zbuf[i, :] = jnp.zeros((F1,), jnp.float32)
            return carry
        lax.fori_loop(0, nt, zb, 0)
        pltpu.sync_copy(zbuf, acc_sp.at[pl.ds(sid * nt, nt)])
        plsc.subcore_barrier()

        ngrp = rpt // nb
        for b in range(nb):
            pltpu.async_copy(h_hbm.at[srcb.at[b]], hbuf.at[b], gsem.at[b])

        def grp(g, carry):
            for b in range(nb):
                j = g * nb + b
                pltpu.make_async_copy(
                    h_hbm.at[srcb.at[b]], hbuf.at[b], gsem.at[b]).wait()

                @pl.when(g > 0)
                def _():
                    pltpu.make_async_copy(
                        sbuf.at[b], acc_sp.at[dstb.at[0]], ssem.at[b]).wait()

                for q in range(CH // 16):
                    n16 = normb[j, pl.ds(q * 16, 16)]
                    for l in range(16):
                        r = q * 16 + l
                        sbuf[b, r, :] = hbuf[b, r, :] * n16[l]
                pltpu.async_copy(sbuf.at[b], acc_sp.at[dstb.at[j]],
                                 ssem.at[b], add=True)

                @pl.when(j + nb < rpt)
                def _():
                    pltpu.async_copy(h_hbm.at[srcb.at[j + nb]], hbuf.at[b],
                                     gsem.at[b])
            return carry
        lax.fori_loop(0, ngrp, grp, 0)
        for b in range(nb):
            pltpu.make_async_copy(
                sbuf.at[b], acc_sp.at[dstb.at[0]], ssem.at[b]).wait()
        plsc.subcore_barrier()
        pltpu.sync_copy(acc_sp.at[pl.ds(sid * nt, nt)],
                        out_hbm.at[pl.ds(cid * npad + sid * nt, nt)])

    return agg


def _mm1_body(x_ref, w_ref, o_ref):
    o_ref[...] = jnp.dot(x_ref[...], w_ref[...],
                         preferred_element_type=jnp.float32)


def _ew1_body(p_ref, s_ref, c_ref, o_ref):
    o_ref[...] = jnp.maximum(
        (p_ref[0] + p_ref[1]) * s_ref[...] + c_ref[...], 0.0)


def _ew2_body(p_ref, w2_ref, s2_ref, c2_ref, w3_ref, o_ref):
    t = p_ref[0] + p_ref[1]
    u = jnp.maximum(
        jnp.dot(t, w2_ref[...], preferred_element_type=jnp.float32)
        * s2_ref[...] + c2_ref[...], 0.0)
    h3 = jnp.dot(u, w3_ref[...], preferred_element_type=jnp.float32)
    o_ref[...] = jnp.concatenate([h3, jnp.zeros_like(h3)], axis=1)


def _ew3_body(p_ref, b3_ref, wl_ref, bl_ref, o_ref):
    t = (p_ref[0] + p_ref[1])[:, :8] + b3_ref[...]
    o_ref[...] = jax.nn.sigmoid(
        jnp.dot(t, wl_ref[...], preferred_element_type=jnp.float32)
        + bl_ref[...])


def _tc(body, out_shape, *args):
    return pl.pallas_call(
        body, out_shape=jax.ShapeDtypeStruct(out_shape, jnp.float32))(*args)


def kernel(x, edge_index, edge_attr, W1, b1, g1, be1, W2, b2, g2, be2,
           W3, b3, Wl, bl):
    f32 = jnp.float32
    n, _ = x.shape
    e = edge_index.shape[1]

    nt = -(-n // 256) * 16
    npad = 16 * nt
    ef = -(-(e + n) // (256 * CH)) * (256 * CH)
    rows = ef // CH
    rpt32 = rows // 32
    rpt16 = rows // 16
    pad = ef - e - n

    idt = edge_index.dtype
    loops = jnp.arange(n, dtype=idt)
    zpad_i = n + (jnp.arange(pad, dtype=idt) % (npad - n))
    src2 = jnp.concatenate([edge_index[0], loops, zpad_i]).reshape(rows, CH)
    dst2 = jnp.concatenate([edge_index[1], loops, zpad_i]).reshape(rows, CH)
    w2d = jnp.concatenate(
        [edge_attr, jnp.ones((n,), f32), jnp.zeros((pad,), f32)]
    ).reshape(rows, CH)
    xp = jnp.pad(x, ((0, npad - n), (0, 0)))

    bn_s = 1.0 / jnp.sqrt(jnp.float32(1.0 + EPS))
    s1 = (g1 * bn_s).reshape(1, F1)
    c1 = (b1 * g1 * bn_s + be1).reshape(1, F1)
    s2 = (g2 * bn_s).reshape(1, 4)
    c2 = (b2 * g2 * bn_s + be2).reshape(1, 4)

    norm2 = _make_precompute(rows, rpt16, npad, nt)(src2, dst2, w2d)

    agg = _make_agg(rows, rpt32, npad, nt)

    h1 = _tc(_mm1_body, (npad, F1), xp, W1.T)
    p1 = agg(h1, src2, dst2, norm2).reshape(2, npad, F1)
    z1 = _tc(_ew1_body, (npad, F1), p1, s1, c1)
    p2 = agg(z1, src2, dst2, norm2).reshape(2, npad, F1)
    h3 = _tc(_ew2_body, (npad, F1), p2, W2.T, s2, c2, W3.T)
    p3 = agg(h3, src2, dst2, norm2).reshape(2, npad, F1)
    y = _tc(_ew3_body, (npad, 8), p3, b3.reshape(1, 8), Wl.T, bl.reshape(1, 8))
    return y[:n]


# speedup vs baseline: 65.5940x; 1.3944x over previous
import functools

import jax
import jax.numpy as jnp
from jax import lax
from jax.experimental import pallas as pl
from jax.experimental.pallas import tpu as pltpu
from jax.experimental.pallas import tpu_sc as plsc

CH = 80
EPS = 1e-5
F1 = 16


def _rsqrt_newton(d):
    ib = plsc.bitcast(d, jnp.int32)
    y = plsc.bitcast(jnp.int32(0x5F3759DF) - (ib >> 1), jnp.float32)
    for _ in range(3):
        y = y * (1.5 - 0.5 * d * y * y)
    return y


def _make_precompute(rows, rpt, npad, nt):
    mesh = plsc.VectorSubcoreMesh(core_axis_name="c", subcore_axis_name="s")

    @functools.partial(
        pl.kernel,
        mesh=mesh,
        out_type=(
            jax.ShapeDtypeStruct((rows, CH), jnp.float32),
            jax.ShapeDtypeStruct((npad, F1), jnp.float32),
        ),
        compiler_params=pltpu.CompilerParams(
            needs_layout_passes=False, use_tc_tiling_on_sc=False),
        scratch_types=[
            pltpu.VMEM((rpt, CH), jnp.int32),
            pltpu.VMEM((rpt, CH), jnp.int32),
            pltpu.VMEM((rpt, CH), jnp.float32),
            pltpu.VMEM((rpt, CH), jnp.float32),
            pltpu.VMEM((npad,), jnp.float32),
            pltpu.VMEM((nt,), jnp.float32),
            pltpu.VMEM((nt, F1), jnp.float32),
            pltpu.VMEM_SHARED((npad,), jnp.float32),
            pltpu.VMEM_SHARED((npad,), jnp.float32),
            pltpu.SemaphoreType.DMA,
        ],
    )
    def pre(src_hbm, dst_hbm, w_hbm, norm_hbm, self_hbm,
            srcb, dstb, wb, normb, dinvb, degb, selfb, deg_sp, dinv_sp,
            dsem):
        cid = lax.axis_index("c")
        sid = lax.axis_index("s")

        @pl.when(cid == 0)
        def _stage():
            t0 = sid * rpt
            pltpu.sync_copy(src_hbm.at[pl.ds(t0, rpt)], srcb)
            pltpu.sync_copy(dst_hbm.at[pl.ds(t0, rpt)], dstb)
            pltpu.sync_copy(w_hbm.at[pl.ds(t0, rpt)], wb)

            def zb(i, carry):
                degb[pl.ds(i * 16, 16)] = jnp.zeros((16,), jnp.float32)
                return carry
            lax.fori_loop(0, nt // 16, zb, 0)
            pltpu.sync_copy(degb, deg_sp.at[pl.ds(sid * nt, nt)])

        plsc.subcore_barrier()

        @pl.when(cid == 0)
        def _deg():
            k = 10
            def grp(g, carry):
                for b in range(k):
                    pltpu.async_copy(wb.at[g * k + b],
                                     deg_sp.at[dstb.at[g * k + b]], dsem,
                                     add=True)
                for _ in range(k):
                    pltpu.make_async_copy(
                        wb.at[0], deg_sp.at[dstb.at[0]], dsem).wait()
                return carry
            lax.fori_loop(0, rpt // k, grp, 0)

        plsc.subcore_barrier()

        @pl.when(cid == 0)
        def _dinv():
            pltpu.sync_copy(deg_sp.at[pl.ds(sid * nt, nt)], degb)

            def nw(i, carry):
                d = degb[pl.ds(i * 16, 16)] + 1.0
                y = _rsqrt_newton(d)
                degb[pl.ds(i * 16, 16)] = y
                for l in range(16):
                    selfb[i * 16 + l, :] = jnp.broadcast_to(
                        (y * y)[l], (F1,))
                return carry
            lax.fori_loop(0, nt // 16, nw, 0)
            pltpu.sync_copy(degb, dinv_sp.at[pl.ds(sid * nt, nt)])
            pltpu.sync_copy(selfb, self_hbm.at[pl.ds(sid * nt, nt)])

        plsc.subcore_barrier()

        @pl.when(cid == 0)
        def _norm():
            pltpu.sync_copy(dinv_sp, dinvb)

            def nc(j, carry):
                for g in range(CH // 16):
                    s16 = srcb[j, pl.ds(g * 16, 16)]
                    d16 = dstb[j, pl.ds(g * 16, 16)]
                    w16 = wb[j, pl.ds(g * 16, 16)]
                    nv = (plsc.load_gather(dinvb, [s16]) * w16
                          * plsc.load_gather(dinvb, [d16]))
                    normb[j, pl.ds(g * 16, 16)] = nv
                return carry
            lax.fori_loop(0, rpt, nc, 0)
            pltpu.sync_copy(normb, norm_hbm.at[pl.ds(sid * rpt, rpt)])

    return pre


def _make_agg(rows, rpt, npad, nt):
    mesh = plsc.VectorSubcoreMesh(core_axis_name="c", subcore_axis_name="s")

    nb = 5

    @functools.partial(
        pl.kernel,
        mesh=mesh,
        out_type=jax.ShapeDtypeStruct((2 * npad, F1), jnp.float32),
        compiler_params=pltpu.CompilerParams(
            needs_layout_passes=False, use_tc_tiling_on_sc=False),
        scratch_types=[
            pltpu.VMEM((rpt, CH), jnp.int32),
            pltpu.VMEM((rpt, CH), jnp.int32),
            pltpu.VMEM((rpt, CH), jnp.float32),
            pltpu.VMEM((nb, CH, F1), jnp.float32),
            pltpu.VMEM((nb, CH, F1), jnp.float32),
            pltpu.VMEM((nt, F1), jnp.float32),
            pltpu.VMEM_SHARED((npad, F1), jnp.float32),
            pltpu.SemaphoreType.DMA((nb,)),
            pltpu.SemaphoreType.DMA((nb,)),
        ],
    )
    def agg(h_hbm, src_hbm, dst_hbm, norm_hbm, out_hbm,
            srcb, dstb, normb, hbuf, sbuf, zbuf, acc_sp, gsem, ssem):
        cid = lax.axis_index("c")
        sid = lax.axis_index("s")
        wid = cid * 16 + sid
        r0 = wid * rpt
        pltpu.sync_copy(src_hbm.at[pl.ds(r0, rpt)], srcb)
        pltpu.sync_copy(dst_hbm.at[pl.ds(r0, rpt)], dstb)
        pltpu.sync_copy(norm_hbm.at[pl.ds(r0, rpt)], normb)

        def zb(i, carry):
            zbuf[i, :] = jnp.zeros((F1,), jnp.float32)
            return carry
        lax.fori_loop(0, nt, zb, 0)
        pltpu.sync_copy(zbuf, acc_sp.at[pl.ds(sid * nt, nt)])
        plsc.subcore_barrier()

        ngrp = rpt // nb
        for b in range(nb):
            pltpu.async_copy(h_hbm.at[srcb.at[b]], hbuf.at[b], gsem.at[b])

        def grp(g, carry):
            for b in range(nb):
                j = g * nb + b
                pltpu.make_async_copy(
                    h_hbm.at[srcb.at[b]], hbuf.at[b], gsem.at[b]).wait()

                @pl.when(g > 0)
                def _():
                    pltpu.make_async_copy(
                        sbuf.at[b], acc_sp.at[dstb.at[0]], ssem.at[b]).wait()

                for q in range(CH // 16):
                    n16 = normb[j, pl.ds(q * 16, 16)]
                    for l in range(16):
                        r = q * 16 + l
                        sbuf[b, r, :] = hbuf[b, r, :] * n16[l]
                pltpu.async_copy(sbuf.at[b], acc_sp.at[dstb.at[j]],
                                 ssem.at[b], add=True)

                @pl.when(j + nb < rpt)
                def _():
                    pltpu.async_copy(h_hbm.at[srcb.at[j + nb]], hbuf.at[b],
                                     gsem.at[b])
            return carry
        lax.fori_loop(0, ngrp, grp, 0)
        for b in range(nb):
            pltpu.make_async_copy(
                sbuf.at[b], acc_sp.at[dstb.at[0]], ssem.at[b]).wait()
        plsc.subcore_barrier()
        pltpu.sync_copy(acc_sp.at[pl.ds(sid * nt, nt)],
                        out_hbm.at[pl.ds(cid * npad + sid * nt, nt)])

    return agg


def _mm1_body(x_ref, w_ref, o_ref):
    o_ref[...] = jnp.dot(x_ref[...], w_ref[...],
                         preferred_element_type=jnp.float32)


def _ew1_body(p_ref, h_ref, se_ref, s_ref, c_ref, o_ref):
    rp = p_ref.shape[0] // 2
    a = p_ref[:rp] + p_ref[rp:] + se_ref[...] * h_ref[...]
    o_ref[...] = jnp.maximum(a * s_ref[...] + c_ref[...], 0.0)


def _ew2_body(p_ref, z_ref, se_ref, w2_ref, s2_ref, c2_ref, w3_ref, o_ref):
    rp = p_ref.shape[0] // 2
    t = p_ref[:rp] + p_ref[rp:] + se_ref[...] * z_ref[...]
    u = jnp.maximum(
        jnp.dot(t, w2_ref[...], preferred_element_type=jnp.float32)
        * s2_ref[...] + c2_ref[...], 0.0)
    o_ref[...] = jnp.dot(u, w3_ref[...], preferred_element_type=jnp.float32)


def _ew3_body(p_ref, h_ref, se_ref, b3_ref, wl_ref, bl_ref, o_ref):
    rp = p_ref.shape[0] // 2
    t = p_ref[:rp] + p_ref[rp:] + se_ref[...] * h_ref[...] + b3_ref[...]
    o_ref[...] = jax.nn.sigmoid(
        jnp.dot(t, wl_ref[...], preferred_element_type=jnp.float32)
        + bl_ref[...])


def _tc(body, out_shape, *args):
    return pl.pallas_call(
        body, out_shape=jax.ShapeDtypeStruct(out_shape, jnp.float32))(*args)


def kernel(x, edge_index, edge_attr, W1, b1, g1, be1, W2, b2, g2, be2,
           W3, b3, Wl, bl):
    f32 = jnp.float32
    n, fin = x.shape
    e = edge_index.shape[1]

    nt = -(-n // 256) * 16
    npad = 16 * nt
    rows = e // CH
    rpt32 = rows // 32
    rpt16 = rows // 16
    rp = npad // 8

    src2 = edge_index[0].reshape(rows, CH)
    dst2 = edge_index[1].reshape(rows, CH)
    w2d = edge_attr.reshape(rows, CH)
    xp = jnp.pad(x, ((0, npad - n), (0, 0)))

    eye8 = jnp.eye(8, dtype=f32)
    bn_s = 1.0 / jnp.sqrt(jnp.float32(1.0 + EPS))
    s1 = jnp.tile(g1 * bn_s, 8).reshape(1, 128)
    c1 = jnp.tile(b1 * g1 * bn_s + be1, 8).reshape(1, 128)
    s2 = jnp.tile(g2 * bn_s, 8).reshape(1, 32)
    c2 = jnp.tile(b2 * g2 * bn_s + be2, 8).reshape(1, 32)
    w1bd = jnp.kron(eye8, W1.T)
    w2bd = jnp.kron(eye8, W2.T)
    w3p = jnp.pad(W3.T, ((0, 0), (0, 8)))
    w3bd = jnp.kron(eye8, w3p)
    wlp = jnp.pad(Wl.T, ((0, 8), (0, 0)))
    wlbd = jnp.kron(eye8, wlp)
    b3t = jnp.tile(jnp.pad(b3, (0, 8)), 8).reshape(1, 128)
    blt = jnp.tile(bl, 8).reshape(1, 64)

    norm2, selfexp = _make_precompute(rows, rpt16, npad, nt)(src2, dst2, w2d)
    sep = selfexp.reshape(rp, 128)

    agg = _make_agg(rows, rpt32, npad, nt)

    h1p = _tc(_mm1_body, (rp, 128), xp.reshape(rp, 8 * fin), w1bd)
    p1 = agg(h1p.reshape(npad, F1), src2, dst2, norm2)
    z1 = _tc(_ew1_body, (rp, 128), p1.reshape(2 * rp, 128), h1p, sep, s1, c1)
    p2 = agg(z1.reshape(npad, F1), src2, dst2, norm2)
    h3 = _tc(_ew2_body, (rp, 128), p2.reshape(2 * rp, 128), z1, sep,
             w2bd, s2, c2, w3bd)
    p3 = agg(h3.reshape(npad, F1), src2, dst2, norm2)
    yp = _tc(_ew3_body, (rp, 64), p3.reshape(2 * rp, 128), h3, sep,
             b3t, wlbd, blt)
    return yp.reshape(npad, 8)[:n]


# speedup vs baseline: 74.6406x; 1.1379x over previous
import functools

import jax
import jax.numpy as jnp
from jax import lax
from jax.experimental import pallas as pl
from jax.experimental.pallas import tpu as pltpu
from jax.experimental.pallas import tpu_sc as plsc

CH = 80
EPS = 1e-5
F1 = 16


def _rsqrt_newton(d):
    ib = plsc.bitcast(d, jnp.int32)
    y = plsc.bitcast(jnp.int32(0x5F3759DF) - (ib >> 1), jnp.float32)
    for _ in range(3):
        y = y * (1.5 - 0.5 * d * y * y)
    return y


def _make_precompute(rows, rpt, npad, nt):
    mesh = plsc.VectorSubcoreMesh(core_axis_name="c", subcore_axis_name="s")

    @functools.partial(
        pl.kernel,
        mesh=mesh,
        out_type=(
            jax.ShapeDtypeStruct((rows, CH), jnp.float32),
            jax.ShapeDtypeStruct((npad, F1), jnp.float32),
        ),
        compiler_params=pltpu.CompilerParams(
            needs_layout_passes=False, use_tc_tiling_on_sc=False),
        scratch_types=[
            pltpu.VMEM((rpt, CH), jnp.int32),
            pltpu.VMEM((rpt, CH), jnp.int32),
            pltpu.VMEM((rpt, CH), jnp.float32),
            pltpu.VMEM((rpt, CH), jnp.float32),
            pltpu.VMEM((npad,), jnp.float32),
            pltpu.VMEM((nt,), jnp.float32),
            pltpu.VMEM((nt, F1), jnp.float32),
            pltpu.VMEM_SHARED((npad,), jnp.float32),
            pltpu.VMEM_SHARED((npad,), jnp.float32),
            pltpu.SemaphoreType.DMA,
        ],
    )
    def pre(src_hbm, dst_hbm, w_hbm, norm_hbm, self_hbm,
            srcb, dstb, wb, normb, dinvb, degb, selfb, deg_sp, dinv_sp,
            dsem):
        cid = lax.axis_index("c")
        sid = lax.axis_index("s")

        @pl.when(cid == 0)
        def _stage():
            t0 = sid * rpt
            pltpu.sync_copy(src_hbm.at[pl.ds(t0, rpt)], srcb)
            pltpu.sync_copy(dst_hbm.at[pl.ds(t0, rpt)], dstb)
            pltpu.sync_copy(w_hbm.at[pl.ds(t0, rpt)], wb)

            def zb(i, carry):
                degb[pl.ds(i * 16, 16)] = jnp.zeros((16,), jnp.float32)
                return carry
            lax.fori_loop(0, nt // 16, zb, 0)
            pltpu.sync_copy(degb, deg_sp.at[pl.ds(sid * nt, nt)])

        plsc.subcore_barrier()

        @pl.when(cid == 0)
        def _deg():
            k = 10
            def grp(g, carry):
                for b in range(k):
                    pltpu.async_copy(wb.at[g * k + b],
                                     deg_sp.at[dstb.at[g * k + b]], dsem,
                                     add=True)
                for _ in range(k):
                    pltpu.make_async_copy(
                        wb.at[0], deg_sp.at[dstb.at[0]], dsem).wait()
                return carry
            lax.fori_loop(0, rpt // k, grp, 0)

        plsc.subcore_barrier()

        @pl.when(cid == 0)
        def _dinv():
            pltpu.sync_copy(deg_sp.at[pl.ds(sid * nt, nt)], degb)

            def nw(i, carry):
                d = degb[pl.ds(i * 16, 16)] + 1.0
                y = _rsqrt_newton(d)
                degb[pl.ds(i * 16, 16)] = y
                for l in range(16):
                    selfb[i * 16 + l, :] = jnp.broadcast_to(
                        (y * y)[l], (F1,))
                return carry
            lax.fori_loop(0, nt // 16, nw, 0)
            pltpu.sync_copy(degb, dinv_sp.at[pl.ds(sid * nt, nt)])
            pltpu.sync_copy(selfb, self_hbm.at[pl.ds(sid * nt, nt)])

        plsc.subcore_barrier()

        @pl.when(cid == 0)
        def _norm():
            pltpu.sync_copy(dinv_sp, dinvb)

            def nc(j, carry):
                for g in range(CH // 16):
                    s16 = srcb[j, pl.ds(g * 16, 16)]
                    d16 = dstb[j, pl.ds(g * 16, 16)]
                    w16 = wb[j, pl.ds(g * 16, 16)]
                    nv = (plsc.load_gather(dinvb, [s16]) * w16
                          * plsc.load_gather(dinvb, [d16]))
                    normb[j, pl.ds(g * 16, 16)] = nv
                return carry
            lax.fori_loop(0, rpt, nc, 0)
            pltpu.sync_copy(normb, norm_hbm.at[pl.ds(sid * rpt, rpt)])

    return pre


def _make_agg(rows, rpt, npad, nt):
    mesh = plsc.VectorSubcoreMesh(core_axis_name="c", subcore_axis_name="s")

    nb = 5

    @functools.partial(
        pl.kernel,
        mesh=mesh,
        out_type=jax.ShapeDtypeStruct((2 * npad, F1), jnp.float32),
        compiler_params=pltpu.CompilerParams(
            needs_layout_passes=False, use_tc_tiling_on_sc=False),
        scratch_types=[
            pltpu.VMEM((rpt, CH), jnp.int32),
            pltpu.VMEM((rpt, CH), jnp.int32),
            pltpu.VMEM((rpt, CH), jnp.float32),
            pltpu.VMEM((nb, CH, F1), jnp.float32),
            pltpu.VMEM((nb, CH, F1), jnp.float32),
            pltpu.VMEM((nt, F1), jnp.float32),
            pltpu.VMEM_SHARED((npad, F1), jnp.float32),
            pltpu.VMEM_SHARED((npad, F1), jnp.float32),
            pltpu.SemaphoreType.DMA((nb,)),
            pltpu.SemaphoreType.DMA((nb,)),
        ],
    )
    def agg(h_hbm, src_hbm, dst_hbm, norm_hbm, out_hbm,
            srcb, dstb, normb, hbuf, sbuf, zbuf, acc_sp, h_sp, gsem, ssem):
        cid = lax.axis_index("c")
        sid = lax.axis_index("s")
        wid = cid * 16 + sid
        r0 = wid * rpt
        pltpu.sync_copy(src_hbm.at[pl.ds(r0, rpt)], srcb)
        pltpu.sync_copy(dst_hbm.at[pl.ds(r0, rpt)], dstb)
        pltpu.sync_copy(norm_hbm.at[pl.ds(r0, rpt)], normb)
        pltpu.sync_copy(h_hbm.at[pl.ds(sid * nt, nt)],
                        h_sp.at[pl.ds(sid * nt, nt)])

        def zb(i, carry):
            zbuf[i, :] = jnp.zeros((F1,), jnp.float32)
            return carry
        lax.fori_loop(0, nt, zb, 0)
        pltpu.sync_copy(zbuf, acc_sp.at[pl.ds(sid * nt, nt)])
        plsc.subcore_barrier()

        ngrp = rpt // nb
        for b in range(nb):
            pltpu.async_copy(h_sp.at[srcb.at[b]], hbuf.at[b], gsem.at[b])

        def grp(g, carry):
            for b in range(nb):
                j = g * nb + b
                pltpu.make_async_copy(
                    h_sp.at[srcb.at[b]], hbuf.at[b], gsem.at[b]).wait()

                @pl.when(g > 0)
                def _():
                    pltpu.make_async_copy(
                        sbuf.at[b], acc_sp.at[dstb.at[0]], ssem.at[b]).wait()

                for q in range(CH // 16):
                    n16 = normb[j, pl.ds(q * 16, 16)]
                    for l in range(16):
                        r = q * 16 + l
                        sbuf[b, r, :] = hbuf[b, r, :] * n16[l]
                pltpu.async_copy(sbuf.at[b], acc_sp.at[dstb.at[j]],
                                 ssem.at[b], add=True)

                @pl.when(j + nb < rpt)
                def _():
                    pltpu.async_copy(h_sp.at[srcb.at[j + nb]], hbuf.at[b],
                                     gsem.at[b])
            return carry
        lax.fori_loop(0, ngrp, grp, 0)
        for b in range(nb):
            pltpu.make_async_copy(
                sbuf.at[b], acc_sp.at[dstb.at[0]], ssem.at[b]).wait()
        plsc.subcore_barrier()
        pltpu.sync_copy(acc_sp.at[pl.ds(sid * nt, nt)],
                        out_hbm.at[pl.ds(cid * npad + sid * nt, nt)])

    return agg


def _mm1_body(x_ref, w_ref, o_ref):
    o_ref[...] = jnp.dot(x_ref[...], w_ref[...],
                         preferred_element_type=jnp.float32)


def _ew1_body(p_ref, h_ref, se_ref, s_ref, c_ref, o_ref):
    rp = p_ref.shape[0] // 2
    a = p_ref[:rp] + p_ref[rp:] + se_ref[...] * h_ref[...]
    o_ref[...] = jnp.maximum(a * s_ref[...] + c_ref[...], 0.0)


def _ew2_body(p_ref, z_ref, se_ref, w2_ref, s2_ref, c2_ref, w3_ref, o_ref):
    rp = p_ref.shape[0] // 2
    t = p_ref[:rp] + p_ref[rp:] + se_ref[...] * z_ref[...]
    u = jnp.maximum(
        jnp.dot(t, w2_ref[...], preferred_element_type=jnp.float32)
        * s2_ref[...] + c2_ref[...], 0.0)
    o_ref[...] = jnp.dot(u, w3_ref[...], preferred_element_type=jnp.float32)


def _ew3_body(p_ref, h_ref, se_ref, b3_ref, wl_ref, bl_ref, o_ref):
    rp = p_ref.shape[0] // 2
    t = p_ref[:rp] + p_ref[rp:] + se_ref[...] * h_ref[...] + b3_ref[...]
    o_ref[...] = jax.nn.sigmoid(
        jnp.dot(t, wl_ref[...], preferred_element_type=jnp.float32)
        + bl_ref[...])


def _tc(body, out_shape, *args):
    return pl.pallas_call(
        body, out_shape=jax.ShapeDtypeStruct(out_shape, jnp.float32))(*args)


def kernel(x, edge_index, edge_attr, W1, b1, g1, be1, W2, b2, g2, be2,
           W3, b3, Wl, bl):
    f32 = jnp.float32
    n, fin = x.shape
    e = edge_index.shape[1]

    nt = -(-n // 256) * 16
    npad = 16 * nt
    rows = e // CH
    rpt32 = rows // 32
    rpt16 = rows // 16
    rp = npad // 8

    src2 = edge_index[0].reshape(rows, CH)
    dst2 = edge_index[1].reshape(rows, CH)
    w2d = edge_attr.reshape(rows, CH)
    xp = jnp.pad(x, ((0, npad - n), (0, 0)))

    eye8 = jnp.eye(8, dtype=f32)
    bn_s = 1.0 / jnp.sqrt(jnp.float32(1.0 + EPS))
    s1 = jnp.tile(g1 * bn_s, 8).reshape(1, 128)
    c1 = jnp.tile(b1 * g1 * bn_s + be1, 8).reshape(1, 128)
    s2 = jnp.tile(g2 * bn_s, 8).reshape(1, 32)
    c2 = jnp.tile(b2 * g2 * bn_s + be2, 8).reshape(1, 32)
    w1bd = jnp.kron(eye8, W1.T)
    w2bd = jnp.kron(eye8, W2.T)
    w3p = jnp.pad(W3.T, ((0, 0), (0, 8)))
    w3bd = jnp.kron(eye8, w3p)
    wlp = jnp.pad(Wl.T, ((0, 8), (0, 0)))
    wlbd = jnp.kron(eye8, wlp)
    b3t = jnp.tile(jnp.pad(b3, (0, 8)), 8).reshape(1, 128)
    blt = jnp.tile(bl, 8).reshape(1, 64)

    norm2, selfexp = _make_precompute(rows, rpt16, npad, nt)(src2, dst2, w2d)
    sep = selfexp.reshape(rp, 128)

    agg = _make_agg(rows, rpt32, npad, nt)

    h1p = _tc(_mm1_body, (rp, 128), xp.reshape(rp, 8 * fin), w1bd)
    p1 = agg(h1p.reshape(npad, F1), src2, dst2, norm2)
    z1 = _tc(_ew1_body, (rp, 128), p1.reshape(2 * rp, 128), h1p, sep, s1, c1)
    p2 = agg(z1.reshape(npad, F1), src2, dst2, norm2)
    h3 = _tc(_ew2_body, (rp, 128), p2.reshape(2 * rp, 128), z1, sep,
             w2bd, s2, c2, w3bd)
    p3 = agg(h3.reshape(npad, F1), src2, dst2, norm2)
    yp = _tc(_ew3_body, (rp, 64), p3.reshape(2 * rp, 128), h3, sep,
             b3t, wlbd, blt)
    return yp.reshape(npad, 8)[:n]


# speedup vs baseline: 75.4217x; 1.0105x over previous
import functools

import jax
import jax.numpy as jnp
from jax import lax
from jax.experimental import pallas as pl
from jax.experimental.pallas import tpu as pltpu
from jax.experimental.pallas import tpu_sc as plsc

CH = 128
EPS = 1e-5
F1 = 16


def _rsqrt_newton(d):
    ib = plsc.bitcast(d, jnp.int32)
    y = plsc.bitcast(jnp.int32(0x5F3759DF) - (ib >> 1), jnp.float32)
    for _ in range(3):
        y = y * (1.5 - 0.5 * d * y * y)
    return y


def _make_precompute(rows, rpt, npad, nt):
    mesh = plsc.VectorSubcoreMesh(core_axis_name="c", subcore_axis_name="s")

    @functools.partial(
        pl.kernel,
        mesh=mesh,
        out_type=(
            jax.ShapeDtypeStruct((rows, CH), jnp.float32),
            jax.ShapeDtypeStruct((npad, F1), jnp.float32),
        ),
        compiler_params=pltpu.CompilerParams(
            needs_layout_passes=False, use_tc_tiling_on_sc=False),
        scratch_types=[
            pltpu.VMEM((rpt, 2, CH), jnp.int32),
            pltpu.VMEM((rpt, CH), jnp.float32),
            pltpu.VMEM((rpt, CH), jnp.float32),
            pltpu.VMEM((npad,), jnp.float32),
            pltpu.VMEM((nt,), jnp.float32),
            pltpu.VMEM((nt, F1), jnp.float32),
            pltpu.VMEM_SHARED((npad,), jnp.float32),
            pltpu.VMEM_SHARED((npad,), jnp.float32),
            pltpu.SemaphoreType.DMA,
        ],
    )
    def pre(ei_hbm, w_hbm, norm_hbm, self_hbm,
            eib, wb, normb, dinvb, degb, selfb, deg_sp, dinv_sp, dsem):
        cid = lax.axis_index("c")
        sid = lax.axis_index("s")

        @pl.when(cid == 0)
        def _stage():
            t0 = sid * rpt
            pltpu.sync_copy(ei_hbm.at[pl.ds(t0, rpt)], eib)
            pltpu.sync_copy(w_hbm.at[pl.ds(t0, rpt)], wb)

            def zb(i, carry):
                degb[pl.ds(i * 16, 16)] = jnp.zeros((16,), jnp.float32)
                return carry
            lax.fori_loop(0, nt // 16, zb, 0)
            pltpu.sync_copy(degb, deg_sp.at[pl.ds(sid * nt, nt)])

        plsc.subcore_barrier()

        @pl.when(cid == 0)
        def _deg():
            k = 10
            def grp(g, carry):
                for b in range(k):
                    pltpu.async_copy(wb.at[g * k + b],
                                     deg_sp.at[eib.at[g * k + b, 1]], dsem,
                                     add=True)
                for _ in range(k):
                    pltpu.make_async_copy(
                        wb.at[0], deg_sp.at[eib.at[0, 1]], dsem).wait()
                return carry
            lax.fori_loop(0, rpt // k, grp, 0)

        plsc.subcore_barrier()

        @pl.when(cid == 0)
        def _dinv():
            pltpu.sync_copy(deg_sp.at[pl.ds(sid * nt, nt)], degb)

            def nw(i, carry):
                d = degb[pl.ds(i * 16, 16)] + 1.0
                y = _rsqrt_newton(d)
                degb[pl.ds(i * 16, 16)] = y
                for l in range(16):
                    selfb[i * 16 + l, :] = jnp.broadcast_to(
                        (y * y)[l], (F1,))
                return carry
            lax.fori_loop(0, nt // 16, nw, 0)
            pltpu.sync_copy(degb, dinv_sp.at[pl.ds(sid * nt, nt)])
            pltpu.sync_copy(selfb, self_hbm.at[pl.ds(sid * nt, nt)])

        plsc.subcore_barrier()

        @pl.when(cid == 0)
        def _norm():
            pltpu.sync_copy(dinv_sp, dinvb)

            def nc(j, carry):
                for g in range(CH // 16):
                    s16 = eib[j, 0, pl.ds(g * 16, 16)]
                    d16 = eib[j, 1, pl.ds(g * 16, 16)]
                    w16 = wb[j, pl.ds(g * 16, 16)]
                    nv = (plsc.load_gather(dinvb, [s16]) * w16
                          * plsc.load_gather(dinvb, [d16]))
                    normb[j, pl.ds(g * 16, 16)] = nv
                return carry
            lax.fori_loop(0, rpt, nc, 0)
            pltpu.sync_copy(normb, norm_hbm.at[pl.ds(sid * rpt, rpt)])

    return pre


def _make_agg(rows, rpt, npad, nt):
    mesh = plsc.VectorSubcoreMesh(core_axis_name="c", subcore_axis_name="s")

    nb = 5

    @functools.partial(
        pl.kernel,
        mesh=mesh,
        out_type=jax.ShapeDtypeStruct((2 * npad, F1), jnp.float32),
        compiler_params=pltpu.CompilerParams(
            needs_layout_passes=False, use_tc_tiling_on_sc=False),
        scratch_types=[
            pltpu.VMEM((rpt, 2, CH), jnp.int32),
            pltpu.VMEM((rpt, CH), jnp.float32),
            pltpu.VMEM((nb, CH, F1), jnp.float32),
            pltpu.VMEM((nb, CH, F1), jnp.float32),
            pltpu.VMEM((nt, F1), jnp.float32),
            pltpu.VMEM_SHARED((npad, F1), jnp.float32),
            pltpu.VMEM_SHARED((npad, F1), jnp.float32),
            pltpu.SemaphoreType.DMA((nb,)),
            pltpu.SemaphoreType.DMA((nb,)),
        ],
    )
    def agg(h_hbm, ei_hbm, norm_hbm, out_hbm,
            eib, normb, hbuf, sbuf, zbuf, acc_sp, h_sp, gsem, ssem):
        cid = lax.axis_index("c")
        sid = lax.axis_index("s")
        wid = cid * 16 + sid
        r0 = wid * rpt
        pltpu.sync_copy(ei_hbm.at[pl.ds(r0, rpt)], eib)
        pltpu.sync_copy(norm_hbm.at[pl.ds(r0, rpt)], normb)
        pltpu.sync_copy(h_hbm.at[pl.ds(sid * nt, nt)],
                        h_sp.at[pl.ds(sid * nt, nt)])

        def zb(i, carry):
            zbuf[i, :] = jnp.zeros((F1,), jnp.float32)
            return carry
        lax.fori_loop(0, nt, zb, 0)
        pltpu.sync_copy(zbuf, acc_sp.at[pl.ds(sid * nt, nt)])
        plsc.subcore_barrier()

        ngrp = rpt // nb
        for b in range(nb):
            pltpu.async_copy(h_sp.at[eib.at[b, 0]], hbuf.at[b], gsem.at[b])

        def grp(g, carry):
            for b in range(nb):
                j = g * nb + b
                pltpu.make_async_copy(
                    h_sp.at[eib.at[b, 0]], hbuf.at[b], gsem.at[b]).wait()

                @pl.when(g > 0)
                def _():
                    pltpu.make_async_copy(
                        sbuf.at[b], acc_sp.at[eib.at[0, 1]],
                        ssem.at[b]).wait()

                for q in range(CH // 16):
                    n16 = normb[j, pl.ds(q * 16, 16)]
                    for l in range(16):
                        r = q * 16 + l
                        sbuf[b, r, :] = hbuf[b, r, :] * n16[l]
                pltpu.async_copy(sbuf.at[b], acc_sp.at[eib.at[j, 1]],
                                 ssem.at[b], add=True)

                @pl.when(j + nb < rpt)
                def _():
                    pltpu.async_copy(h_sp.at[eib.at[j + nb, 0]], hbuf.at[b],
                                     gsem.at[b])
            return carry
        lax.fori_loop(0, ngrp, grp, 0)
        for b in range(nb):
            pltpu.make_async_copy(
                sbuf.at[b], acc_sp.at[eib.at[0, 1]], ssem.at[b]).wait()
        plsc.subcore_barrier()
        pltpu.sync_copy(acc_sp.at[pl.ds(sid * nt, nt)],
                        out_hbm.at[pl.ds(cid * npad + sid * nt, nt)])

    return agg


def _mm1_body(x_ref, w_ref, o_ref):
    o_ref[...] = jnp.dot(x_ref[...], w_ref[...],
                         preferred_element_type=jnp.float32)


def _ew1_body(p_ref, h_ref, se_ref, s_ref, c_ref, o_ref):
    rp = p_ref.shape[0] // 2
    a = p_ref[:rp] + p_ref[rp:] + se_ref[...] * h_ref[...]
    o_ref[...] = jnp.maximum(a * s_ref[...] + c_ref[...], 0.0)


def _ew2_body(p_ref, z_ref, se_ref, w2_ref, s2_ref, c2_ref, w3_ref, o_ref):
    rp = p_ref.shape[0] // 2
    t = p_ref[:rp] + p_ref[rp:] + se_ref[...] * z_ref[...]
    u = jnp.maximum(
        jnp.dot(t, w2_ref[...], preferred_element_type=jnp.float32)
        * s2_ref[...] + c2_ref[...], 0.0)
    o_ref[...] = jnp.dot(u, w3_ref[...], preferred_element_type=jnp.float32)


def _ew3_body(p_ref, h_ref, se_ref, b3_ref, wl_ref, bl_ref, o_ref):
    rp = p_ref.shape[0] // 2
    t = p_ref[:rp] + p_ref[rp:] + se_ref[...] * h_ref[...] + b3_ref[...]
    o_ref[...] = jax.nn.sigmoid(
        jnp.dot(t, wl_ref[...], preferred_element_type=jnp.float32)
        + bl_ref[...])


def _tc(body, out_shape, *args):
    return pl.pallas_call(
        body, out_shape=jax.ShapeDtypeStruct(out_shape, jnp.float32))(*args)


def kernel(x, edge_index, edge_attr, W1, b1, g1, be1, W2, b2, g2, be2,
           W3, b3, Wl, bl):
    f32 = jnp.float32
    n, fin = x.shape
    e = edge_index.shape[1]

    nt = -(-n // 256) * 16
    npad = 16 * nt
    rows_e = e // CH
    rows = ((rows_e + 159) // 160) * 160
    rpt32 = rows // 32
    rpt16 = rows // 16
    padr = rows - rows_e
    rp = npad // 8

    idt = edge_index.dtype
    ei3 = edge_index.reshape(2, rows_e, CH).transpose(1, 0, 2)
    padidx = (n + (jnp.arange(padr * CH, dtype=idt) % (npad - n))
              ).reshape(padr, 1, CH)
    ei = jnp.concatenate([ei3, jnp.broadcast_to(padidx, (padr, 2, CH))])
    w2d = jnp.concatenate([edge_attr.reshape(rows_e, CH),
                           jnp.zeros((padr, CH), f32)])
    xp = jnp.pad(x, ((0, npad - n), (0, 0)))

    eye8 = jnp.eye(8, dtype=f32)
    bn_s = 1.0 / jnp.sqrt(jnp.float32(1.0 + EPS))
    s1 = jnp.tile(g1 * bn_s, 8).reshape(1, 128)
    c1 = jnp.tile(b1 * g1 * bn_s + be1, 8).reshape(1, 128)
    s2 = jnp.tile(g2 * bn_s, 8).reshape(1, 32)
    c2 = jnp.tile(b2 * g2 * bn_s + be2, 8).reshape(1, 32)
    w1bd = jnp.kron(eye8, W1.T)
    w2bd = jnp.kron(eye8, W2.T)
    w3p = jnp.pad(W3.T, ((0, 0), (0, 8)))
    w3bd = jnp.kron(eye8, w3p)
    wlp = jnp.pad(Wl.T, ((0, 8), (0, 0)))
    wlbd = jnp.kron(eye8, wlp)
    b3t = jnp.tile(jnp.pad(b3, (0, 8)), 8).reshape(1, 128)
    blt = jnp.tile(bl, 8).reshape(1, 64)

    norm2, selfexp = _make_precompute(rows, rpt16, npad, nt)(ei, w2d)
    sep = selfexp.reshape(rp, 128)

    agg = _make_agg(rows, rpt32, npad, nt)

    h1p = _tc(_mm1_body, (rp, 128), xp.reshape(rp, 8 * fin), w1bd)
    p1 = agg(h1p.reshape(npad, F1), ei, norm2)
    z1 = _tc(_ew1_body, (rp, 128), p1.reshape(2 * rp, 128), h1p, sep, s1, c1)
    p2 = agg(z1.reshape(npad, F1), ei, norm2)
    h3 = _tc(_ew2_body, (rp, 128), p2.reshape(2 * rp, 128), z1, sep,
             w2bd, s2, c2, w3bd)
    p3 = agg(h3.reshape(npad, F1), ei, norm2)
    yp = _tc(_ew3_body, (rp, 64), p3.reshape(2 * rp, 128), h3, sep,
             b3t, wlbd, blt)
    return yp.reshape(npad, 8)[:n]
